# baseline (speedup 1.0000x reference)
import functools

import jax
import jax.numpy as jnp
from jax import lax
from jax.experimental import pallas as pl
from jax.experimental.pallas import tpu as pltpu
from jax.experimental.pallas import tpu_sc as plsc

_N = 10000
_E = 320000
_F = 128
_H = 128
_C = 64

_NC = 2
_NS = 16
_NW = _NC * _NS
_RPT = _N // _NS


def _make_sc_aggregate(D, with_count, nbuf, grp, full_idx, K):
    mesh = plsc.VectorSubcoreMesh(core_axis_name="c", subcore_axis_name="s",
                                  num_cores=_NC, num_subcores=_NS)
    out_type = [jax.ShapeDtypeStruct((_N, D), jnp.float32)] * 2
    if with_count:
        out_type += [jax.ShapeDtypeStruct((_N, 16), jnp.float32)] * 2
    cpw = _E // (_NW * K)
    islab = cpw if full_idx else grp
    scratch = [
        pltpu.VMEM((islab, K), jnp.int32),
        pltpu.VMEM((islab, K), jnp.int32),
    ] + [pltpu.VMEM((K, D), jnp.float32) for _ in range(nbuf)] + [
        pltpu.VMEM_SHARED((_N, D), jnp.float32),
    ] + [pltpu.SemaphoreType.DMA for _ in range(nbuf)]
    if with_count:
        scratch += [
            pltpu.VMEM((K, 16), jnp.float32),
            pltpu.VMEM_SHARED((_N, 16), jnp.float32),
            pltpu.SemaphoreType.DMA,
        ]

    def body(table, src_hbm, dst_hbm, *rest):
        n_out = 4 if with_count else 2
        outs, rest = rest[:n_out], rest[n_out:]
        a0_out, a1_out = outs[0], outs[1]
        src_v, dst_v = rest[0], rest[1]
        rows = rest[2:2 + nbuf]
        acc_sh = rest[2 + nbuf]
        sems = rest[3 + nbuf:3 + 2 * nbuf]
        if with_count:
            c0_out, c1_out = outs[2], outs[3]
            ones_v, cnt_sh, csem = rest[3 + 2 * nbuf:]
        rows_v = rows[0]
        cid = lax.axis_index("c")
        sid = lax.axis_index("s")
        wid = sid * _NC + cid

        cpr = D // 16

        def zero_rows(i, _):
            rows_v[i // cpr, pl.ds((i % cpr) * 16, 16)] = jnp.zeros(
                (16,), jnp.float32)
            return 0

        lax.fori_loop(0, K * cpr, zero_rows, 0)
        base = sid * _RPT
        zch = 125 if K % 125 == 0 else 25
        for i in range(_RPT // zch):
            pltpu.sync_copy(rows_v.at[pl.ds(0, zch)],
                            acc_sh.at[pl.ds(base + i * zch, zch)])

        if with_count:
            def fill(val):
                def f(i, _):
                    ones_v[i, pl.ds(0, 16)] = jnp.full((16,), val, jnp.float32)
                    return 0
                return f
            lax.fori_loop(0, K, fill(0.0), 0)
            for i in range(_RPT // zch):
                pltpu.sync_copy(ones_v.at[pl.ds(0, zch)],
                                cnt_sh.at[pl.ds(base + i * zch, zch)])
            lax.fori_loop(0, K, fill(1.0), 0)

        if full_idx:
            pltpu.sync_copy(src_hbm.at[pl.ds(wid * cpw, cpw)], src_v)
            pltpu.sync_copy(dst_hbm.at[pl.ds(wid * cpw, cpw)], dst_v)

        plsc.subcore_barrier()
        pre = min(nbuf - 1, grp)

        def group(g, _):
            if full_idx:
                def row(j):
                    return g * grp + j
            else:
                gb = wid * cpw + g * grp
                pltpu.sync_copy(src_hbm.at[pl.ds(gb, grp)], src_v)
                pltpu.sync_copy(dst_hbm.at[pl.ds(gb, grp)], dst_v)

                def row(j):
                    return j

            pend = [None] * grp
            for j in range(pre):
                pend[j] = pltpu.async_copy(table.at[src_v.at[row(j)]],
                                           rows[j % nbuf], sems[j % nbuf])
            cds = []
            for j in range(grp):
                pend[j].wait()
                nxt = j + pre
                if nxt < grp:
                    pend[nxt] = pltpu.async_copy(
                        table.at[src_v.at[row(nxt)]],
                        rows[nxt % nbuf], sems[nxt % nbuf])
                pltpu.sync_copy(rows[j % nbuf], acc_sh.at[dst_v.at[row(j)]],
                                add=True)
                if with_count:
                    cds.append(pltpu.async_copy(
                        ones_v, cnt_sh.at[dst_v.at[row(j)]], csem, add=True))
            for d in cds:
                d.wait()
            return 0

        lax.fori_loop(0, cpw // grp, group, 0)
        plsc.subcore_barrier()

        ob = pl.multiple_of(sid * 624, 8)

        def copy_out(acc_out, cnt_out):
            pltpu.sync_copy(acc_sh.at[pl.ds(ob, 624)],
                            acc_out.at[pl.ds(ob, 624)])
            if with_count:
                pltpu.sync_copy(cnt_sh.at[pl.ds(ob, 624)],
                                cnt_out.at[pl.ds(ob, 624)])

            @pl.when(sid == _NS - 1)
            def _():
                pltpu.sync_copy(acc_sh.at[pl.ds(9984, 16)],
                                acc_out.at[pl.ds(9984, 16)])
                if with_count:
                    pltpu.sync_copy(cnt_sh.at[pl.ds(9984, 16)],
                                    cnt_out.at[pl.ds(9984, 16)])

        @pl.when(cid == 0)
        def _():
            copy_out(a0_out, c0_out if with_count else None)

        @pl.when(cid == 1)
        def _():
            copy_out(a1_out, c1_out if with_count else None)

    return pl.kernel(
        body, out_type=tuple(out_type), mesh=mesh, scratch_types=scratch,
        compiler_params=pltpu.CompilerParams(use_tc_tiling_on_sc=False))


_K1 = 80
_K2 = 125
_agg_l1 = _make_sc_aggregate(_F, with_count=True, nbuf=3, grp=25,
                             full_idx=False, K=_K1)
_agg_l2 = _make_sc_aggregate(_C, with_count=False, nbuf=6, grp=20,
                             full_idx=True, K=_K2)

_BN = 1000


def _mid_body(a0, a1, c0, c1, x, w1l, w1r, b1, w2l, w2r, b2, y2, r2):
    s = a0[...] + a1[...]
    cnt = (c0[...] + c1[...])[:, 0:1]
    inv = 1.0 / jnp.maximum(cnt, 1.0)
    h = jnp.maximum(
        jnp.dot(s * inv, w1l[...], preferred_element_type=jnp.float32)
        + jnp.dot(x[...], w1r[...], preferred_element_type=jnp.float32)
        + b1[...], 0.0)
    y2[...] = jnp.dot(h, w2l[...], preferred_element_type=jnp.float32)
    r2[...] = jnp.dot(h, w2r[...], preferred_element_type=jnp.float32) + b2[...]


def _fin_body(a0, a1, c0, c1, r2, out):
    cnt = (c0[...] + c1[...])[:, 0:1]
    inv = 1.0 / jnp.maximum(cnt, 1.0)
    t = (a0[...] + a1[...]) * inv + r2[...]
    m = jnp.max(t, axis=1, keepdims=True)
    lse = jnp.log(jnp.sum(jnp.exp(t - m), axis=1, keepdims=True))
    out[...] = (t - m) - lse


def _row_spec(d):
    return pl.BlockSpec((_BN, d), lambda i: (i, 0))


def _full_spec(r, c):
    return pl.BlockSpec((r, c), lambda i: (0, 0))


_mid = pl.pallas_call(
    _mid_body,
    grid=(_N // _BN,),
    in_specs=[
        _row_spec(_F), _row_spec(_F), _row_spec(16), _row_spec(16),
        _row_spec(_F),
        _full_spec(_F, _H), _full_spec(_F, _H), _full_spec(1, _H),
        _full_spec(_H, _C), _full_spec(_H, _C), _full_spec(1, _C),
    ],
    out_specs=[_row_spec(_C), _row_spec(_C)],
    out_shape=[
        jax.ShapeDtypeStruct((_N, _C), jnp.float32),
        jax.ShapeDtypeStruct((_N, _C), jnp.float32),
    ],
)

_fin = pl.pallas_call(
    _fin_body,
    grid=(_N // _BN,),
    in_specs=[
        _row_spec(_C), _row_spec(_C), _row_spec(16), _row_spec(16),
        _row_spec(_C),
    ],
    out_specs=_row_spec(_C),
    out_shape=jax.ShapeDtypeStruct((_N, _C), jnp.float32),
)


def kernel(x, edge_index, W1_l, W1_r, b1, W2_l, W2_r, b2):
    src1 = edge_index[0].reshape(_E // _K1, _K1)
    dst1 = edge_index[1].reshape(_E // _K1, _K1)
    src2 = edge_index[0].reshape(_E // _K2, _K2)
    dst2 = edge_index[1].reshape(_E // _K2, _K2)
    a0, a1, c0, c1 = _agg_l1(x, src1, dst1)
    y2, r2 = _mid(a0, a1, c0, c1, x, W1_l, W1_r,
                  b1.reshape(1, _H), W2_l, W2_r, b2.reshape(1, _C))
    g0, g1 = _agg_l2(y2, src2, dst2)
    return _fin(g0, g1, c0, c1, r2)


# speedup vs baseline: 1.0048x; 1.0048x over previous
import functools

import jax
import jax.numpy as jnp
from jax import lax
from jax.experimental import pallas as pl
from jax.experimental.pallas import tpu as pltpu
from jax.experimental.pallas import tpu_sc as plsc

_N = 10000
_E = 320000
_F = 128
_H = 128
_C = 64

_NC = 2
_NS = 16
_NW = _NC * _NS
_RPT = _N // _NS


def _make_sc_aggregate(D, with_count, nbuf, grp, full_idx, K):
    mesh = plsc.VectorSubcoreMesh(core_axis_name="c", subcore_axis_name="s",
                                  num_cores=_NC, num_subcores=_NS)
    out_type = [jax.ShapeDtypeStruct((_N, D), jnp.float32)] * 2
    if with_count:
        out_type += [jax.ShapeDtypeStruct((_N, 16), jnp.float32)] * 2
    cpw = _E // (_NW * K)
    islab = cpw if full_idx else grp
    scratch = [
        pltpu.VMEM((islab, K), jnp.int32),
        pltpu.VMEM((islab, K), jnp.int32),
    ] + [pltpu.VMEM((K, D), jnp.float32) for _ in range(nbuf)] + [
        pltpu.VMEM_SHARED((_N, D), jnp.float32),
    ] + [pltpu.SemaphoreType.DMA for _ in range(nbuf)]
    if with_count:
        scratch += [
            pltpu.VMEM((K, 16), jnp.float32),
            pltpu.VMEM_SHARED((_N, 16), jnp.float32),
            pltpu.SemaphoreType.DMA,
        ]

    def body(table, src_hbm, dst_hbm, *rest):
        n_out = 4 if with_count else 2
        outs, rest = rest[:n_out], rest[n_out:]
        a0_out, a1_out = outs[0], outs[1]
        src_v, dst_v = rest[0], rest[1]
        rows = rest[2:2 + nbuf]
        acc_sh = rest[2 + nbuf]
        sems = rest[3 + nbuf:3 + 2 * nbuf]
        if with_count:
            c0_out, c1_out = outs[2], outs[3]
            ones_v, cnt_sh, csem = rest[3 + 2 * nbuf:]
        rows_v = rows[0]
        cid = lax.axis_index("c")
        sid = lax.axis_index("s")
        wid = sid * _NC + cid

        cpr = D // 16

        def zero_rows(i, _):
            rows_v[i // cpr, pl.ds((i % cpr) * 16, 16)] = jnp.zeros(
                (16,), jnp.float32)
            return 0

        lax.fori_loop(0, K * cpr, zero_rows, 0)
        base = sid * _RPT
        zch = 125 if K % 125 == 0 else 25
        for i in range(_RPT // zch):
            pltpu.sync_copy(rows_v.at[pl.ds(0, zch)],
                            acc_sh.at[pl.ds(base + i * zch, zch)])

        if with_count:
            def fill(val):
                def f(i, _):
                    ones_v[i, pl.ds(0, 16)] = jnp.full((16,), val, jnp.float32)
                    return 0
                return f
            lax.fori_loop(0, K, fill(0.0), 0)
            for i in range(_RPT // zch):
                pltpu.sync_copy(ones_v.at[pl.ds(0, zch)],
                                cnt_sh.at[pl.ds(base + i * zch, zch)])
            lax.fori_loop(0, K, fill(1.0), 0)

        if full_idx:
            pltpu.sync_copy(src_hbm.at[pl.ds(wid * cpw, cpw)], src_v)
            pltpu.sync_copy(dst_hbm.at[pl.ds(wid * cpw, cpw)], dst_v)

        plsc.subcore_barrier()
        pre = min(nbuf - 1, grp)

        def group(g, _):
            if full_idx:
                def row(j):
                    return g * grp + j
            else:
                gb = wid * cpw + g * grp
                pltpu.sync_copy(src_hbm.at[pl.ds(gb, grp)], src_v)
                pltpu.sync_copy(dst_hbm.at[pl.ds(gb, grp)], dst_v)

                def row(j):
                    return j

            pend = [None] * grp
            for j in range(pre):
                pend[j] = pltpu.async_copy(table.at[src_v.at[row(j)]],
                                           rows[j % nbuf], sems[j % nbuf])
            cds = []
            for j in range(grp):
                pend[j].wait()
                nxt = j + pre
                if nxt < grp:
                    pend[nxt] = pltpu.async_copy(
                        table.at[src_v.at[row(nxt)]],
                        rows[nxt % nbuf], sems[nxt % nbuf])
                pltpu.sync_copy(rows[j % nbuf], acc_sh.at[dst_v.at[row(j)]],
                                add=True)
                if with_count:
                    cds.append(pltpu.async_copy(
                        ones_v, cnt_sh.at[dst_v.at[row(j)]], csem, add=True))
            for d in cds:
                d.wait()
            return 0

        lax.fori_loop(0, cpw // grp, group, 0)
        plsc.subcore_barrier()

        ob = pl.multiple_of(sid * 624, 8)

        def copy_out(acc_out, cnt_out):
            pltpu.sync_copy(acc_sh.at[pl.ds(ob, 624)],
                            acc_out.at[pl.ds(ob, 624)])
            if with_count:
                pltpu.sync_copy(cnt_sh.at[pl.ds(ob, 624)],
                                cnt_out.at[pl.ds(ob, 624)])

            @pl.when(sid == _NS - 1)
            def _():
                pltpu.sync_copy(acc_sh.at[pl.ds(9984, 16)],
                                acc_out.at[pl.ds(9984, 16)])
                if with_count:
                    pltpu.sync_copy(cnt_sh.at[pl.ds(9984, 16)],
                                    cnt_out.at[pl.ds(9984, 16)])

        @pl.when(cid == 0)
        def _():
            copy_out(a0_out, c0_out if with_count else None)

        @pl.when(cid == 1)
        def _():
            copy_out(a1_out, c1_out if with_count else None)

    return pl.kernel(
        body, out_type=tuple(out_type), mesh=mesh, scratch_types=scratch,
        compiler_params=pltpu.CompilerParams(use_tc_tiling_on_sc=False))


_K1 = 80
_K2 = 125
_agg_l1 = _make_sc_aggregate(_F, with_count=True, nbuf=3, grp=25,
                             full_idx=False, K=_K1)
_agg_l2 = _make_sc_aggregate(_C, with_count=False, nbuf=4, grp=20,
                             full_idx=True, K=_K2)

_BN = 1000


def _mid_body(a0, a1, c0, c1, x, w1l, w1r, b1, w2l, w2r, b2, y2, r2):
    s = a0[...] + a1[...]
    cnt = (c0[...] + c1[...])[:, 0:1]
    inv = 1.0 / jnp.maximum(cnt, 1.0)
    h = jnp.maximum(
        jnp.dot(s * inv, w1l[...], preferred_element_type=jnp.float32)
        + jnp.dot(x[...], w1r[...], preferred_element_type=jnp.float32)
        + b1[...], 0.0)
    y2[...] = jnp.dot(h, w2l[...], preferred_element_type=jnp.float32)
    r2[...] = jnp.dot(h, w2r[...], preferred_element_type=jnp.float32) + b2[...]


def _fin_body(a0, a1, c0, c1, r2, out):
    cnt = (c0[...] + c1[...])[:, 0:1]
    inv = 1.0 / jnp.maximum(cnt, 1.0)
    t = (a0[...] + a1[...]) * inv + r2[...]
    m = jnp.max(t, axis=1, keepdims=True)
    lse = jnp.log(jnp.sum(jnp.exp(t - m), axis=1, keepdims=True))
    out[...] = (t - m) - lse


def _row_spec(d):
    return pl.BlockSpec((_BN, d), lambda i: (i, 0))


def _full_spec(r, c):
    return pl.BlockSpec((r, c), lambda i: (0, 0))


_mid = pl.pallas_call(
    _mid_body,
    grid=(_N // _BN,),
    in_specs=[
        _row_spec(_F), _row_spec(_F), _row_spec(16), _row_spec(16),
        _row_spec(_F),
        _full_spec(_F, _H), _full_spec(_F, _H), _full_spec(1, _H),
        _full_spec(_H, _C), _full_spec(_H, _C), _full_spec(1, _C),
    ],
    out_specs=[_row_spec(_C), _row_spec(_C)],
    out_shape=[
        jax.ShapeDtypeStruct((_N, _C), jnp.float32),
        jax.ShapeDtypeStruct((_N, _C), jnp.float32),
    ],
)

_fin = pl.pallas_call(
    _fin_body,
    grid=(_N // _BN,),
    in_specs=[
        _row_spec(_C), _row_spec(_C), _row_spec(16), _row_spec(16),
        _row_spec(_C),
    ],
    out_specs=_row_spec(_C),
    out_shape=jax.ShapeDtypeStruct((_N, _C), jnp.float32),
)


def kernel(x, edge_index, W1_l, W1_r, b1, W2_l, W2_r, b2):
    src1 = edge_index[0].reshape(_E // _K1, _K1)
    dst1 = edge_index[1].reshape(_E // _K1, _K1)
    src2 = edge_index[0].reshape(_E // _K2, _K2)
    dst2 = edge_index[1].reshape(_E // _K2, _K2)
    a0, a1, c0, c1 = _agg_l1(x, src1, dst1)
    y2, r2 = _mid(a0, a1, c0, c1, x, W1_l, W1_r,
                  b1.reshape(1, _H), W2_l, W2_r, b2.reshape(1, _C))
    g0, g1 = _agg_l2(y2, src2, dst2)
    return _fin(g0, g1, c0, c1, r2)


# speedup vs baseline: 1.0087x; 1.0038x over previous
import functools

import jax
import jax.numpy as jnp
from jax import lax
from jax.experimental import pallas as pl
from jax.experimental.pallas import tpu as pltpu
from jax.experimental.pallas import tpu_sc as plsc

_N = 10000
_E = 320000
_F = 128
_H = 128
_C = 64

_NC = 2
_NS = 16
_NW = _NC * _NS
_RPT = _N // _NS


def _make_sc_aggregate(D, with_count, nbuf, grp, full_idx, K):
    mesh = plsc.VectorSubcoreMesh(core_axis_name="c", subcore_axis_name="s",
                                  num_cores=_NC, num_subcores=_NS)
    out_type = [jax.ShapeDtypeStruct((_N, D), jnp.float32)] * 2
    if with_count:
        out_type += [jax.ShapeDtypeStruct((_N, 16), jnp.float32)] * 2
    cpw = _E // (_NW * K)
    islab = cpw if full_idx else grp
    scratch = [
        pltpu.VMEM((islab, K), jnp.int32),
        pltpu.VMEM((islab, K), jnp.int32),
    ] + [pltpu.VMEM((K, D), jnp.float32) for _ in range(nbuf)] + [
        pltpu.VMEM_SHARED((_N, D), jnp.float32),
    ] + [pltpu.SemaphoreType.DMA for _ in range(2 * nbuf)]
    if with_count:
        scratch += [
            pltpu.VMEM((K, 16), jnp.float32),
            pltpu.VMEM_SHARED((_N, 16), jnp.float32),
            pltpu.SemaphoreType.DMA,
        ]

    def body(table, src_hbm, dst_hbm, *rest):
        n_out = 4 if with_count else 2
        outs, rest = rest[:n_out], rest[n_out:]
        a0_out, a1_out = outs[0], outs[1]
        src_v, dst_v = rest[0], rest[1]
        rows = rest[2:2 + nbuf]
        acc_sh = rest[2 + nbuf]
        sems = rest[3 + nbuf:3 + 2 * nbuf]
        ssems = rest[3 + 2 * nbuf:3 + 3 * nbuf]
        if with_count:
            c0_out, c1_out = outs[2], outs[3]
            ones_v, cnt_sh, csem = rest[3 + 3 * nbuf:]
        rows_v = rows[0]
        cid = lax.axis_index("c")
        sid = lax.axis_index("s")
        wid = sid * _NC + cid

        cpr = D // 16

        def zero_rows(i, _):
            rows_v[i // cpr, pl.ds((i % cpr) * 16, 16)] = jnp.zeros(
                (16,), jnp.float32)
            return 0

        lax.fori_loop(0, K * cpr, zero_rows, 0)
        base = sid * _RPT
        zch = 125 if K % 125 == 0 else 25
        for i in range(_RPT // zch):
            pltpu.sync_copy(rows_v.at[pl.ds(0, zch)],
                            acc_sh.at[pl.ds(base + i * zch, zch)])

        if with_count:
            def fill(val):
                def f(i, _):
                    ones_v[i, pl.ds(0, 16)] = jnp.full((16,), val, jnp.float32)
                    return 0
                return f
            lax.fori_loop(0, K, fill(0.0), 0)
            for i in range(_RPT // zch):
                pltpu.sync_copy(ones_v.at[pl.ds(0, zch)],
                                cnt_sh.at[pl.ds(base + i * zch, zch)])
            lax.fori_loop(0, K, fill(1.0), 0)

        if full_idx:
            pltpu.sync_copy(src_hbm.at[pl.ds(wid * cpw, cpw)], src_v)
            pltpu.sync_copy(dst_hbm.at[pl.ds(wid * cpw, cpw)], dst_v)

        plsc.subcore_barrier()
        pre = min(nbuf - 1, grp)

        def group(g, _):
            if full_idx:
                def row(j):
                    return g * grp + j
            else:
                gb = wid * cpw + g * grp
                pltpu.sync_copy(src_hbm.at[pl.ds(gb, grp)], src_v)
                pltpu.sync_copy(dst_hbm.at[pl.ds(gb, grp)], dst_v)

                def row(j):
                    return j

            pend = [None] * grp
            asc = [None] * grp
            for j in range(pre):
                pend[j] = pltpu.async_copy(table.at[src_v.at[row(j)]],
                                           rows[j % nbuf], sems[j % nbuf])
            cds = []
            for j in range(grp):
                pend[j].wait()
                nxt = j + pre
                if nxt < grp:
                    if j >= 1 and asc[j - 1] is not None:
                        asc[j - 1].wait()
                        asc[j - 1] = None
                    pend[nxt] = pltpu.async_copy(
                        table.at[src_v.at[row(nxt)]],
                        rows[nxt % nbuf], sems[nxt % nbuf])
                asc[j] = pltpu.async_copy(
                    rows[j % nbuf], acc_sh.at[dst_v.at[row(j)]],
                    ssems[j % nbuf], add=True)
                if with_count:
                    cds.append(pltpu.async_copy(
                        ones_v, cnt_sh.at[dst_v.at[row(j)]], csem, add=True))
            for d in asc:
                if d is not None:
                    d.wait()
            for d in cds:
                d.wait()
            return 0

        lax.fori_loop(0, cpw // grp, group, 0)
        plsc.subcore_barrier()

        ob = pl.multiple_of(sid * 624, 8)

        def copy_out(acc_out, cnt_out):
            pltpu.sync_copy(acc_sh.at[pl.ds(ob, 624)],
                            acc_out.at[pl.ds(ob, 624)])
            if with_count:
                pltpu.sync_copy(cnt_sh.at[pl.ds(ob, 624)],
                                cnt_out.at[pl.ds(ob, 624)])

            @pl.when(sid == _NS - 1)
            def _():
                pltpu.sync_copy(acc_sh.at[pl.ds(9984, 16)],
                                acc_out.at[pl.ds(9984, 16)])
                if with_count:
                    pltpu.sync_copy(cnt_sh.at[pl.ds(9984, 16)],
                                    cnt_out.at[pl.ds(9984, 16)])

        @pl.when(cid == 0)
        def _():
            copy_out(a0_out, c0_out if with_count else None)

        @pl.when(cid == 1)
        def _():
            copy_out(a1_out, c1_out if with_count else None)

    return pl.kernel(
        body, out_type=tuple(out_type), mesh=mesh, scratch_types=scratch,
        compiler_params=pltpu.CompilerParams(use_tc_tiling_on_sc=False))


_K1 = 80
_K2 = 125
_agg_l1 = _make_sc_aggregate(_F, with_count=True, nbuf=3, grp=25,
                             full_idx=False, K=_K1)
_agg_l2 = _make_sc_aggregate(_C, with_count=False, nbuf=4, grp=20,
                             full_idx=True, K=_K2)

_BN = 1000


def _mid_body(a0, a1, c0, c1, x, w1l, w1r, b1, w2l, w2r, b2, y2, r2):
    s = a0[...] + a1[...]
    cnt = (c0[...] + c1[...])[:, 0:1]
    inv = 1.0 / jnp.maximum(cnt, 1.0)
    h = jnp.maximum(
        jnp.dot(s * inv, w1l[...], preferred_element_type=jnp.float32)
        + jnp.dot(x[...], w1r[...], preferred_element_type=jnp.float32)
        + b1[...], 0.0)
    y2[...] = jnp.dot(h, w2l[...], preferred_element_type=jnp.float32)
    r2[...] = jnp.dot(h, w2r[...], preferred_element_type=jnp.float32) + b2[...]


def _fin_body(a0, a1, c0, c1, r2, out):
    cnt = (c0[...] + c1[...])[:, 0:1]
    inv = 1.0 / jnp.maximum(cnt, 1.0)
    t = (a0[...] + a1[...]) * inv + r2[...]
    m = jnp.max(t, axis=1, keepdims=True)
    lse = jnp.log(jnp.sum(jnp.exp(t - m), axis=1, keepdims=True))
    out[...] = (t - m) - lse


def _row_spec(d):
    return pl.BlockSpec((_BN, d), lambda i: (i, 0))


def _full_spec(r, c):
    return pl.BlockSpec((r, c), lambda i: (0, 0))


_mid = pl.pallas_call(
    _mid_body,
    grid=(_N // _BN,),
    in_specs=[
        _row_spec(_F), _row_spec(_F), _row_spec(16), _row_spec(16),
        _row_spec(_F),
        _full_spec(_F, _H), _full_spec(_F, _H), _full_spec(1, _H),
        _full_spec(_H, _C), _full_spec(_H, _C), _full_spec(1, _C),
    ],
    out_specs=[_row_spec(_C), _row_spec(_C)],
    out_shape=[
        jax.ShapeDtypeStruct((_N, _C), jnp.float32),
        jax.ShapeDtypeStruct((_N, _C), jnp.float32),
    ],
)

_fin = pl.pallas_call(
    _fin_body,
    grid=(_N // _BN,),
    in_specs=[
        _row_spec(_C), _row_spec(_C), _row_spec(16), _row_spec(16),
        _row_spec(_C),
    ],
    out_specs=_row_spec(_C),
    out_shape=jax.ShapeDtypeStruct((_N, _C), jnp.float32),
)


def kernel(x, edge_index, W1_l, W1_r, b1, W2_l, W2_r, b2):
    src1 = edge_index[0].reshape(_E // _K1, _K1)
    dst1 = edge_index[1].reshape(_E // _K1, _K1)
    src2 = edge_index[0].reshape(_E // _K2, _K2)
    dst2 = edge_index[1].reshape(_E // _K2, _K2)
    a0, a1, c0, c1 = _agg_l1(x, src1, dst1)
    y2, r2 = _mid(a0, a1, c0, c1, x, W1_l, W1_r,
                  b1.reshape(1, _H), W2_l, W2_r, b2.reshape(1, _C))
    g0, g1 = _agg_l2(y2, src2, dst2)
    return _fin(g0, g1, c0, c1, r2)
